# Initial kernel scaffold; baseline (speedup 1.0000x reference)
import jax
import jax.numpy as jnp
from jax.experimental import pallas as pl


def kernel(x, q, A0, A1, A2, A3, TA, TC):
    raise NotImplementedError("write your pallas kernel here")


# speedup vs baseline: 14.3308x; 14.3308x over previous
import functools

import jax
import jax.numpy as jnp
import numpy as np
from jax import lax
from jax.experimental import pallas as pl
from jax.experimental.pallas import tpu as pltpu
from jax.experimental.pallas import tpu_sc as plsc

VOCAB = 100000
EMBD = 32
STORY = 50
SENT = 20
BS = 1024
HOPS = 3

PAIRS = BS * STORY
NC, NS = 2, 16
NW = NC * NS
PPW = PAIRS // NW
CH = 32
NCHUNK = PPW // CH
IPC = CH * SENT
IDXROWS = IPC // 128
QPW = BS // NW

VC = 2048
NV = (VOCAB + VC - 1) // VC
NVP = NV * VC

_NEG_INF = float("-inf")


def _position_encoding():
    j = np.arange(1, SENT + 1, dtype=np.float32)[:, None]
    k = np.arange(1, EMBD + 1, dtype=np.float32)[None, :]
    pe = 1.0 - j / SENT - (k / EMBD) * (1.0 - 2.0 * j / SENT)
    return jnp.asarray(pe, dtype=jnp.float32)


def _sc_pool_body(xg, qg, a0, a1, a2, a3, pe_hbm,
                  m0, m1, m2, c1, c2, c3, u0,
                  idx_v, rows_v, mbuf, cbuf, pe_v, sem):
    wid = lax.axis_index("s") * NC + lax.axis_index("c")
    pltpu.sync_copy(pe_hbm, pe_v)
    pe_regs = [(pe_v[j, 0:16], pe_v[j, 16:32]) for j in range(SENT)]
    zero = jnp.zeros((16,), jnp.float32)

    def run_table(tbl, idx_src, idx_flat_base, out_pair_base, nchunks,
                  m_out, c_out):
        def chunk_body(i, carry):
            pltpu.sync_copy(idx_src.at[pl.ds(idx_flat_base + i * IPC, IPC)],
                            idx_v)
            cps = [pltpu.async_copy(tbl.at[idx_v.at[pl.ds(r * 128, 128)]],
                                    rows_v.at[pl.ds(r * 128, 128)], sem)
                   for r in range(IDXROWS)]
            for cp in cps:
                cp.wait()

            def pair_body(p, c2_):
                mlo = mhi = clo = chi = zero
                for j in range(SENT):
                    rlo = rows_v[p * SENT + j, 0:16]
                    rhi = rows_v[p * SENT + j, 16:32]
                    if m_out is not None:
                        plo, phi = pe_regs[j]
                        mlo = mlo + rlo * plo
                        mhi = mhi + rhi * phi
                    if c_out is not None:
                        clo = clo + rlo
                        chi = chi + rhi
                if m_out is not None:
                    mbuf[p, 0:16] = mlo
                    mbuf[p, 16:32] = mhi
                if c_out is not None:
                    cbuf[p, 0:16] = clo
                    cbuf[p, 16:32] = chi
                return c2_

            lax.fori_loop(0, CH, pair_body, 0, unroll=False)
            base = out_pair_base + i * CH
            if m_out is not None:
                pltpu.sync_copy(mbuf, m_out.at[pl.ds(base, CH)])
            if c_out is not None:
                pltpu.sync_copy(cbuf, c_out.at[pl.ds(base, CH)])
            return carry

        lax.fori_loop(0, nchunks, chunk_body, 0, unroll=False)

    x_flat_base = wid * (PPW * SENT)
    x_pair_base = wid * PPW
    run_table(a0, xg, x_flat_base, x_pair_base, NCHUNK, m0, None)
    run_table(a1, xg, x_flat_base, x_pair_base, NCHUNK, m1, c1)
    run_table(a2, xg, x_flat_base, x_pair_base, NCHUNK, m2, c2)
    run_table(a3, xg, x_flat_base, x_pair_base, NCHUNK, None, c3)
    run_table(a0, qg, wid * (QPW * SENT), wid * QPW, 1, None, u0)


def _sc_pool(xg, qg, a0, a1, a2, a3, pe):
    f32 = jnp.float32
    out_type = ([jax.ShapeDtypeStruct((PAIRS, EMBD), f32)] * 6
                + [jax.ShapeDtypeStruct((BS, EMBD), f32)])
    mesh = plsc.VectorSubcoreMesh(core_axis_name="c", subcore_axis_name="s")
    kern = pl.kernel(
        _sc_pool_body,
        out_type=out_type,
        mesh=mesh,
        scratch_types=[
            pltpu.VMEM((IPC,), jnp.int32),
            pltpu.VMEM((IPC, EMBD), f32),
            pltpu.VMEM((CH, EMBD), f32),
            pltpu.VMEM((CH, EMBD), f32),
            pltpu.VMEM((SENT, EMBD), f32),
            pltpu.SemaphoreType.DMA,
        ],
        compiler_params=pltpu.CompilerParams(use_tc_tiling_on_sc=False),
    )
    return kern(xg, qg, a0, a1, a2, a3, pe)


def _hops_body(m0, m1, m2, c1, c2, c3, u0, ta, tc, out):
    u = u0[...]
    ta_b = ta[...][None, :, :]
    tc_b = tc[...][None, :, :]
    for mr, cr in ((m0, c1), (m1, c2), (m2, c3)):
        m = mr[...] + ta_b
        logits = jnp.sum(m * u[:, None, :], axis=2)
        p = jax.nn.softmax(logits, axis=1)
        c = cr[...] + tc_b
        u = u + jnp.sum(c * p[:, :, None], axis=1)
    out[...] = u


def _hops(m0, m1, m2, c1, c2, c3, u0, ta, tc):
    blk = 128
    grid = BS // blk
    mem_spec = pl.BlockSpec((blk, STORY, EMBD), lambda i: (i, 0, 0))
    u_spec = pl.BlockSpec((blk, EMBD), lambda i: (i, 0))
    t_spec = pl.BlockSpec((STORY, EMBD), lambda i: (0, 0))
    return pl.pallas_call(
        _hops_body,
        grid=(grid,),
        in_specs=[mem_spec] * 6 + [u_spec, t_spec, t_spec],
        out_specs=u_spec,
        out_shape=jax.ShapeDtypeStruct((BS, EMBD), jnp.float32),
    )(m0, m1, m2, c1, c2, c3, u0, ta, tc)


def _stats_body(u, a3t, lse, macc, sacc):
    i = pl.program_id(0)
    logits = jnp.dot(u[...], a3t[...], preferred_element_type=jnp.float32)
    col = i * VC + lax.broadcasted_iota(jnp.int32, (1, VC), 1)
    valid = col < VOCAB
    logits = jnp.where(valid, logits, _NEG_INF)
    mchunk = jnp.max(logits, axis=1, keepdims=True)

    @pl.when(i == 0)
    def _():
        macc[...] = mchunk
        sacc[...] = jnp.sum(jnp.exp(logits - mchunk), axis=1, keepdims=True)

    @pl.when(i > 0)
    def _():
        mnew = jnp.maximum(macc[...], mchunk)
        sacc[...] = (sacc[...] * jnp.exp(macc[...] - mnew)
                     + jnp.sum(jnp.exp(logits - mnew), axis=1, keepdims=True))
        macc[...] = mnew

    @pl.when(i == NV - 1)
    def _():
        lse[...] = macc[...] + jnp.log(sacc[...])


def _proj_body(u, a3t, lse, out):
    logits = jnp.dot(u[...], a3t[...], preferred_element_type=jnp.float32)
    out[...] = logits - lse[...]


def _projection(u, a3t):
    u_spec = pl.BlockSpec((BS, EMBD), lambda i: (0, 0))
    a3t_spec = pl.BlockSpec((EMBD, VC), lambda i: (0, i))
    lse_spec = pl.BlockSpec((BS, 1), lambda i: (0, 0))
    lse = pl.pallas_call(
        _stats_body,
        grid=(NV,),
        in_specs=[u_spec, a3t_spec],
        out_specs=lse_spec,
        out_shape=jax.ShapeDtypeStruct((BS, 1), jnp.float32),
        scratch_shapes=[pltpu.VMEM((BS, 1), jnp.float32),
                        pltpu.VMEM((BS, 1), jnp.float32)],
    )(u, a3t)
    return pl.pallas_call(
        _proj_body,
        grid=(NV,),
        in_specs=[u_spec, a3t_spec, lse_spec],
        out_specs=pl.BlockSpec((BS, VC), lambda i: (0, i)),
        out_shape=jax.ShapeDtypeStruct((BS, VOCAB), jnp.float32),
    )(u, a3t, lse)


def kernel(x, q, A0, A1, A2, A3, TA, TC):
    pe = _position_encoding()
    xg = x.reshape(PAIRS * SENT)
    qg = q.reshape(BS * SENT)
    m0, m1, m2, c1, c2, c3, u0 = _sc_pool(xg, qg, A0, A1, A2, A3, pe)

    shape3 = (BS, STORY, EMBD)
    u = _hops(m0.reshape(shape3), m1.reshape(shape3), m2.reshape(shape3),
              c1.reshape(shape3), c2.reshape(shape3), c3.reshape(shape3),
              u0, TA.reshape(STORY, EMBD), TC.reshape(STORY, EMBD))

    a3t = jnp.zeros((EMBD, NVP), jnp.float32).at[:, :VOCAB].set(A3.T)
    return _projection(u, a3t)


# speedup vs baseline: 18.4444x; 1.2871x over previous
import functools

import jax
import jax.numpy as jnp
import numpy as np
from jax import lax
from jax.experimental import pallas as pl
from jax.experimental.pallas import tpu as pltpu
from jax.experimental.pallas import tpu_sc as plsc

VOCAB = 100000
EMBD = 32
STORY = 50
SENT = 20
BS = 1024
HOPS = 3

PAIRS = BS * STORY
NC, NS = 2, 16
NW = NC * NS
PPW = PAIRS // NW
CH = 64
NCHUNK = PPW // CH
IPC = CH * SENT
IDXROWS = IPC // 128
QPW = BS // NW
QIPC = QPW * SENT

VC = 2048
NV = (VOCAB + VC - 1) // VC
NVP = NV * VC

_NEG_INF = float("-inf")


def _position_encoding():
    j = np.arange(1, SENT + 1, dtype=np.float32)[:, None]
    k = np.arange(1, EMBD + 1, dtype=np.float32)[None, :]
    pe = 1.0 - j / SENT - (k / EMBD) * (1.0 - 2.0 * j / SENT)
    return jnp.asarray(pe, dtype=jnp.float32)


def _sc_pool_body(xg, qg, a0, a1, a2, a3, pe_hbm,
                  m0, m1, m2, c1, c2, c3, u0,
                  idx_all, rows0, rows1, mbuf, cbuf, pe_v,
                  sem0, sem1):
    wid = lax.axis_index("s") * NC + lax.axis_index("c")
    pltpu.sync_copy(pe_hbm, pe_v)
    pe_regs = [(pe_v[pl.ds(j * EMBD, 16)], pe_v[pl.ds(j * EMBD + 16, 16)])
               for j in range(SENT)]
    zero = jnp.zeros((16,), jnp.float32)
    pltpu.sync_copy(xg.at[pl.ds(wid * (PPW * SENT), PPW * SENT)], idx_all)

    def run_table(tbl, out_pair_base, m_out, c_out):
        def fire(buf, i, sem):
            for r in range(IDXROWS):
                pltpu.async_copy(
                    tbl.at[idx_all.at[pl.ds(i * IPC + r * 128, 128)]],
                    buf.at[pl.ds(r * 128, 128)], sem)

        def drain(buf, sem):
            pltpu.make_async_copy(tbl.at[pl.ds(0, CH * SENT)], buf, sem).wait()

        def compute(buf, i):
            def pair_body(p, carry):
                mlo = mhi = clo = chi = zero
                for j in range(SENT):
                    rlo = buf[p * SENT + j, 0:16]
                    rhi = buf[p * SENT + j, 16:32]
                    if m_out is not None:
                        plo, phi = pe_regs[j]
                        mlo = mlo + rlo * plo
                        mhi = mhi + rhi * phi
                    if c_out is not None:
                        clo = clo + rlo
                        chi = chi + rhi
                if m_out is not None:
                    mbuf[pl.ds(p * EMBD, 16)] = mlo
                    mbuf[pl.ds(p * EMBD + 16, 16)] = mhi
                if c_out is not None:
                    cbuf[pl.ds(p * EMBD, 16)] = clo
                    cbuf[pl.ds(p * EMBD + 16, 16)] = chi
                return carry

            lax.fori_loop(0, CH, pair_body, 0, unroll=False)
            base = (out_pair_base + i * CH) * EMBD
            if m_out is not None:
                pltpu.sync_copy(mbuf, m_out.at[pl.ds(base, CH * EMBD)])
            if c_out is not None:
                pltpu.sync_copy(cbuf, c_out.at[pl.ds(base, CH * EMBD)])

        fire(rows0, 0, sem0)
        fire(rows1, 1, sem1)

        def body(t, carry):
            a = 2 * t
            drain(rows0, sem0)
            compute(rows0, a)

            @pl.when(a + 2 < NCHUNK)
            def _():
                fire(rows0, a + 2, sem0)

            @pl.when(a + 1 < NCHUNK)
            def _():
                drain(rows1, sem1)
                compute(rows1, a + 1)

                @pl.when(a + 3 < NCHUNK)
                def _():
                    fire(rows1, a + 3, sem1)

            return carry

        lax.fori_loop(0, (NCHUNK + 1) // 2, body, 0, unroll=False)

    run_table(a0, wid * PPW, m0, None)
    run_table(a1, wid * PPW, m1, c1)
    run_table(a2, wid * PPW, m2, c2)
    run_table(a3, wid * PPW, None, c3)

    pltpu.sync_copy(qg.at[pl.ds(wid * QIPC, QIPC)],
                    idx_all.at[pl.ds(0, QIPC)])
    for r in range(QIPC // 128):
        pltpu.async_copy(a0.at[idx_all.at[pl.ds(r * 128, 128)]],
                         rows0.at[pl.ds(r * 128, 128)], sem0)
    pltpu.make_async_copy(a0.at[pl.ds(0, QIPC)],
                          rows0.at[pl.ds(0, QIPC)], sem0).wait()

    def q_pair(p, carry):
        clo = chi = zero
        for j in range(SENT):
            clo = clo + rows0[p * SENT + j, 0:16]
            chi = chi + rows0[p * SENT + j, 16:32]
        cbuf[pl.ds(p * EMBD, 16)] = clo
        cbuf[pl.ds(p * EMBD + 16, 16)] = chi
        return carry

    lax.fori_loop(0, QPW, q_pair, 0, unroll=False)
    pltpu.sync_copy(cbuf.at[pl.ds(0, QPW * EMBD)],
                    u0.at[pl.ds(wid * QPW * EMBD, QPW * EMBD)])


def _sc_pool(xg, qg, a0, a1, a2, a3, pe):
    f32 = jnp.float32
    out_type = ([jax.ShapeDtypeStruct((PAIRS * EMBD,), f32)] * 6
                + [jax.ShapeDtypeStruct((BS * EMBD,), f32)])
    mesh = plsc.VectorSubcoreMesh(core_axis_name="c", subcore_axis_name="s")
    kern = pl.kernel(
        _sc_pool_body,
        out_type=out_type,
        mesh=mesh,
        scratch_types=[
            pltpu.VMEM((PPW * SENT,), jnp.int32),
            pltpu.VMEM((IPC, EMBD), f32),
            pltpu.VMEM((IPC, EMBD), f32),
            pltpu.VMEM((CH * EMBD,), f32),
            pltpu.VMEM((CH * EMBD,), f32),
            pltpu.VMEM((SENT * EMBD,), f32),
            pltpu.SemaphoreType.DMA,
            pltpu.SemaphoreType.DMA,
        ],
        compiler_params=pltpu.CompilerParams(use_tc_tiling_on_sc=False),
    )
    return kern(xg, qg, a0, a1, a2, a3, pe)


def _hops_body(m0, m1, m2, c1, c2, c3, u0, ta, tc, out):
    u = u0[...]
    ta_b = ta[...][None, :, :]
    tc_b = tc[...][None, :, :]
    for mr, cr in ((m0, c1), (m1, c2), (m2, c3)):
        m = mr[...] + ta_b
        logits = jnp.sum(m * u[:, None, :], axis=2)
        p = jax.nn.softmax(logits, axis=1)
        c = cr[...] + tc_b
        u = u + jnp.sum(c * p[:, :, None], axis=1)
    out[...] = u


def _hops(m0, m1, m2, c1, c2, c3, u0, ta, tc):
    blk = 128
    grid = BS // blk
    mem_spec = pl.BlockSpec((blk, STORY, EMBD), lambda i: (i, 0, 0))
    u_spec = pl.BlockSpec((blk, EMBD), lambda i: (i, 0))
    t_spec = pl.BlockSpec((STORY, EMBD), lambda i: (0, 0))
    return pl.pallas_call(
        _hops_body,
        grid=(grid,),
        in_specs=[mem_spec] * 6 + [u_spec, t_spec, t_spec],
        out_specs=u_spec,
        out_shape=jax.ShapeDtypeStruct((BS, EMBD), jnp.float32),
    )(m0, m1, m2, c1, c2, c3, u0, ta, tc)


def _stats_body(u, a3t, lse, macc, sacc):
    i = pl.program_id(0)
    logits = jnp.dot(u[...], a3t[...], preferred_element_type=jnp.float32)
    col = i * VC + lax.broadcasted_iota(jnp.int32, (1, VC), 1)
    valid = col < VOCAB
    logits = jnp.where(valid, logits, _NEG_INF)
    mchunk = jnp.max(logits, axis=1, keepdims=True)

    @pl.when(i == 0)
    def _():
        macc[...] = mchunk
        sacc[...] = jnp.sum(jnp.exp(logits - mchunk), axis=1, keepdims=True)

    @pl.when(i > 0)
    def _():
        mnew = jnp.maximum(macc[...], mchunk)
        sacc[...] = (sacc[...] * jnp.exp(macc[...] - mnew)
                     + jnp.sum(jnp.exp(logits - mnew), axis=1, keepdims=True))
        macc[...] = mnew

    @pl.when(i == NV - 1)
    def _():
        lse[...] = macc[...] + jnp.log(sacc[...])


def _proj_body(u, a3t, lse, out):
    logits = jnp.dot(u[...], a3t[...], preferred_element_type=jnp.float32)
    out[...] = logits - lse[...]


def _projection(u, a3t):
    u_spec = pl.BlockSpec((BS, EMBD), lambda i: (0, 0))
    a3t_spec = pl.BlockSpec((EMBD, VC), lambda i: (0, i))
    lse_spec = pl.BlockSpec((BS, 1), lambda i: (0, 0))
    lse = pl.pallas_call(
        _stats_body,
        grid=(NV,),
        in_specs=[u_spec, a3t_spec],
        out_specs=lse_spec,
        out_shape=jax.ShapeDtypeStruct((BS, 1), jnp.float32),
        scratch_shapes=[pltpu.VMEM((BS, 1), jnp.float32),
                        pltpu.VMEM((BS, 1), jnp.float32)],
    )(u, a3t)
    return pl.pallas_call(
        _proj_body,
        grid=(NV,),
        in_specs=[u_spec, a3t_spec, lse_spec],
        out_specs=pl.BlockSpec((BS, VC), lambda i: (0, i)),
        out_shape=jax.ShapeDtypeStruct((BS, VOCAB), jnp.float32),
    )(u, a3t, lse)


def kernel(x, q, A0, A1, A2, A3, TA, TC):
    pe = _position_encoding().reshape(SENT * EMBD)
    xg = x.reshape(PAIRS * SENT)
    qg = q.reshape(BS * SENT)
    m0, m1, m2, c1, c2, c3, u0 = _sc_pool(xg, qg, A0, A1, A2, A3, pe)

    shape3 = (BS, STORY, EMBD)
    u = _hops(m0.reshape(shape3), m1.reshape(shape3), m2.reshape(shape3),
              c1.reshape(shape3), c2.reshape(shape3), c3.reshape(shape3),
              u0.reshape(BS, EMBD),
              TA.reshape(STORY, EMBD), TC.reshape(STORY, EMBD))

    a3t = jnp.zeros((EMBD, NVP), jnp.float32).at[:, :VOCAB].set(A3.T)
    return _projection(u, a3t)
